# baseline (speedup 1.0000x reference)
import functools

import jax
import jax.numpy as jnp
from jax import lax
from jax.experimental import pallas as pl
from jax.experimental.pallas import tpu as pltpu
from jax.experimental.pallas import tpu_sc as plsc

B = 16384
V = 1000001
G = 1001
D = 64
GL = 20

NC = 2
NS = 16
NW = NC * NS
BPW = B // NW
CHUNK = 128
NCHUNK = BPW // CHUNK
LANES = 16
NGRP = CHUNK // LANES

_SC_PARAMS = pltpu.CompilerParams(
    needs_layout_passes=False, use_tc_tiling_on_sc=False)


def _tree_sum(vals):
    vals = list(vals)
    while len(vals) > 1:
        nxt = [vals[i] + vals[i + 1] for i in range(0, len(vals) - 1, 2)]
        if len(vals) % 2:
            nxt.append(vals[-1])
        vals = nxt
    return vals[0]


def _sc_genre(genres_flat, genre_table):
    mesh = plsc.VectorSubcoreMesh(core_axis_name="c", subcore_axis_name="s")

    @functools.partial(
        pl.kernel,
        out_type=jax.ShapeDtypeStruct((B, D), jnp.float32),
        mesh=mesh,
        compiler_params=_SC_PARAMS,
        scratch_types=[
            pltpu.VMEM((G, D + 1), jnp.float32),
            pltpu.VMEM((CHUNK * GL,), jnp.int32),
            pltpu.VMEM((CHUNK, D + 1), jnp.float32),
        ],
    )
    def k(genres_hbm, gtab_hbm, gemb_hbm, gt_v, gids_v, g_v):
        wid = lax.axis_index("s") * NC + lax.axis_index("c")
        base = wid * BPW
        pltpu.sync_copy(gtab_hbm, gt_v.at[:, pl.ds(0, D)])
        iota = lax.iota(jnp.int32, LANES)

        for c in range(NCHUNK):
            cbase = base + c * CHUNK
            pltpu.sync_copy(genres_hbm.at[pl.ds(cbase * GL, CHUNK * GL)], gids_v)

            def grp_body(g, _):
                row16 = g * LANES + iota
                rowg = row16 * GL
                ids = [plsc.load_gather(gids_v, [rowg + l]) for l in range(GL)]
                one = jnp.full((LANES,), 1.0, jnp.float32)
                zero = jnp.full((LANES,), 0.0, jnp.float32)
                cnts = [jnp.where(ids[l] > 0, one, zero) for l in range(GL)]
                cnt = _tree_sum(cnts)
                recip = 1.0 / (cnt + 1e-8)

                @plsc.parallel_loop(0, D, step=1, unroll=4)
                def d_body(dd):
                    dvec = jnp.full((LANES,), dd, jnp.int32)
                    vals = [plsc.load_gather(gt_v, [ids[l], dvec])
                            for l in range(GL)]
                    acc = _tree_sum(vals)
                    plsc.store_scatter(g_v, [row16, dvec], acc * recip)

                return _

            lax.fori_loop(0, NGRP, grp_body, 0)
            pltpu.sync_copy(g_v.at[:, pl.ds(0, D)],
                            gemb_hbm.at[pl.ds(cbase, CHUNK)])

    return k(genres_flat, genre_table)


def _sc_item(item_id, itab_lin):
    mesh = plsc.VectorSubcoreMesh(core_axis_name="c", subcore_axis_name="s")
    FMS = CHUNK + 1

    @functools.partial(
        pl.kernel,
        out_type=jax.ShapeDtypeStruct((B * D,), jnp.float32),
        mesh=mesh,
        compiler_params=_SC_PARAMS,
        scratch_types=[
            pltpu.VMEM((NCHUNK, CHUNK), jnp.int32),
            pltpu.VMEM((D * CHUNK,), jnp.int32),
            pltpu.VMEM((D, FMS), jnp.float32),
            pltpu.VMEM((CHUNK * D,), jnp.float32),
            pltpu.SemaphoreType.DMA,
        ],
    )
    def k(item_id_hbm, itab_hbm, iemb_hbm, ids_v, idx_v, fm_v, irows_v, sem):
        wid = lax.axis_index("s") * NC + lax.axis_index("c")
        base = wid * BPW
        for c in range(NCHUNK):
            pltpu.sync_copy(item_id_hbm.at[pl.ds(base + c * CHUNK, CHUNK)],
                            ids_v.at[c])
        iota = lax.iota(jnp.int32, LANES)

        for c in range(NCHUNK):
            cbase = base + c * CHUNK
            idg = [ids_v[c, pl.ds(g * LANES, LANES)] for g in range(NGRP)]

            @plsc.parallel_loop(0, D, step=1, unroll=2)
            def build(dd):
                off = jnp.full((LANES,), dd * V, jnp.int32)
                for g in range(NGRP):
                    idx_v[pl.ds(dd * CHUNK + g * LANES, LANES)] = idg[g] + off

            def fire(dd, _):
                pltpu.async_copy(
                    itab_hbm.at[idx_v.at[pl.ds(dd * CHUNK, CHUNK)]],
                    fm_v.at[dd, pl.ds(0, CHUNK)], sem)
                return _

            lax.fori_loop(0, D, fire, 0)
            pltpu.make_async_copy(
                itab_hbm.at[pl.ds(0, CHUNK * D)], irows_v, sem).wait()

            @plsc.parallel_loop(0, CHUNK, step=1, unroll=2)
            def trans(j):
                jvec = jnp.full((LANES,), j, jnp.int32)
                for g in range(D // LANES):
                    v = plsc.load_gather(fm_v, [g * LANES + iota, jvec])
                    irows_v[pl.ds(j * D + g * LANES, LANES)] = v

            pltpu.sync_copy(irows_v, iemb_hbm.at[pl.ds(cbase * D, CHUNK * D)])

    return k(item_id, itab_lin)


def _tc_mlp(i_emb, g_emb, cont8, wc8, bc, w1a, w1b, w1c, b1, w2, b2):
    BM = 1024
    grid = (B // BM,)

    def body(i_ref, g_ref, c_ref, wc_ref, bc_ref, w1a_ref, w1b_ref, w1c_ref,
             b1_ref, w2_ref, b2_ref, o_ref):
        ce = jnp.dot(c_ref[...], wc_ref[...], preferred_element_type=jnp.float32)
        ce = jnp.maximum(ce + bc_ref[...], 0.0)
        h = (jnp.dot(i_ref[...], w1a_ref[...], preferred_element_type=jnp.float32)
             + jnp.dot(g_ref[...], w1b_ref[...], preferred_element_type=jnp.float32)
             + jnp.dot(ce, w1c_ref[...], preferred_element_type=jnp.float32)
             + b1_ref[...])
        h = jnp.maximum(h, 0.0)
        out = jnp.dot(h, w2_ref[...], preferred_element_type=jnp.float32) + b2_ref[...]
        nrm = jnp.sqrt(jnp.sum(out * out, axis=1, keepdims=True))
        o_ref[...] = out / jnp.maximum(nrm, 1e-12)

    full = lambda shape: pl.BlockSpec(shape, lambda i: (0, 0))
    return pl.pallas_call(
        body,
        grid=grid,
        in_specs=[
            pl.BlockSpec((BM, D), lambda i: (i, 0)),
            pl.BlockSpec((BM, D), lambda i: (i, 0)),
            pl.BlockSpec((BM, 8), lambda i: (i, 0)),
            full((8, D)),
            full((1, D)),
            full((D, 128)),
            full((D, 128)),
            full((D, 128)),
            full((1, 128)),
            full((128, D)),
            full((1, D)),
        ],
        out_specs=pl.BlockSpec((BM, D), lambda i: (i, 0)),
        out_shape=jax.ShapeDtypeStruct((B, D), jnp.float32),
    )(i_emb, g_emb, cont8, wc8, bc, w1a, w1b, w1c, b1, w2, b2)


def kernel(item_id, tmdb_genres, release_year, avg_rating, revenue,
           item_table, genre_table, w_cont, b_cont, w1, b1, w2, b2):
    item_id = item_id.astype(jnp.int32)
    genres_flat = tmdb_genres.astype(jnp.int32).reshape(-1)
    itab_lin = item_table.T.reshape(-1)

    g_emb = _sc_genre(genres_flat, genre_table)
    i_emb = _sc_item(item_id, itab_lin).reshape(B, D)

    cont = jnp.stack([release_year, avg_rating, revenue], axis=1)
    cont8 = jnp.pad(cont, ((0, 0), (0, 5)))
    wc8 = jnp.pad(w_cont, ((0, 5), (0, 0)))
    w1a, w1b, w1c = w1[:D], w1[D:2 * D], w1[2 * D:]

    return _tc_mlp(i_emb, g_emb, cont8, wc8, b_cont.reshape(1, D),
                   w1a, w1b, w1c, b1.reshape(1, 128), w2, b2.reshape(1, D))


# speedup vs baseline: 7.7183x; 7.7183x over previous
import functools

import jax
import jax.numpy as jnp
from jax import lax
from jax.experimental import pallas as pl
from jax.experimental.pallas import tpu as pltpu
from jax.experimental.pallas import tpu_sc as plsc

B = 16384
V = 1000001
G = 1001
D = 64
GL = 20

NC = 2
NS = 16
NW = NC * NS
BPW = B // NW
CHUNK = 128
NCHUNK = BPW // CHUNK
LANES = 16
NGRP = CHUNK // LANES

_SC_PARAMS = pltpu.CompilerParams(
    needs_layout_passes=False, use_tc_tiling_on_sc=False)


def _tree_sum(vals):
    vals = list(vals)
    while len(vals) > 1:
        nxt = [vals[i] + vals[i + 1] for i in range(0, len(vals) - 1, 2)]
        if len(vals) % 2:
            nxt.append(vals[-1])
        vals = nxt
    return vals[0]


def _sc_genre(genres_flat, genre_table):
    mesh = plsc.VectorSubcoreMesh(core_axis_name="c", subcore_axis_name="s")

    @functools.partial(
        pl.kernel,
        out_type=jax.ShapeDtypeStruct((B, D), jnp.float32),
        mesh=mesh,
        compiler_params=_SC_PARAMS,
        scratch_types=[
            pltpu.VMEM((G, D + 1), jnp.float32),
            pltpu.VMEM((CHUNK * GL,), jnp.int32),
            pltpu.VMEM((CHUNK, D + 1), jnp.float32),
        ],
    )
    def k(genres_hbm, gtab_hbm, gemb_hbm, gt_v, gids_v, g_v):
        wid = lax.axis_index("s") * NC + lax.axis_index("c")
        base = wid * BPW
        pltpu.sync_copy(gtab_hbm, gt_v.at[:, pl.ds(0, D)])
        iota = lax.iota(jnp.int32, LANES)

        for c in range(NCHUNK):
            cbase = base + c * CHUNK
            pltpu.sync_copy(genres_hbm.at[pl.ds(cbase * GL, CHUNK * GL)], gids_v)

            def grp_body(g, _):
                row16 = g * LANES + iota
                rowg = row16 * GL
                ids = [plsc.load_gather(gids_v, [rowg + l]) for l in range(GL)]
                one = jnp.full((LANES,), 1.0, jnp.float32)
                zero = jnp.full((LANES,), 0.0, jnp.float32)
                cnts = [jnp.where(ids[l] > 0, one, zero) for l in range(GL)]
                cnt = _tree_sum(cnts)
                recip = 1.0 / (cnt + 1e-8)

                @plsc.parallel_loop(0, D, step=1, unroll=4)
                def d_body(dd):
                    dvec = jnp.full((LANES,), dd, jnp.int32)
                    vals = [plsc.load_gather(gt_v, [ids[l], dvec])
                            for l in range(GL)]
                    acc = _tree_sum(vals)
                    plsc.store_scatter(g_v, [row16, dvec], acc * recip)

                return _

            lax.fori_loop(0, NGRP, grp_body, 0)
            pltpu.sync_copy(g_v.at[:, pl.ds(0, D)],
                            gemb_hbm.at[pl.ds(cbase, CHUNK)])

    return k(genres_flat, genre_table)


def _sc_item(item_id, item_table):
    mesh = plsc.VectorSubcoreMesh(core_axis_name="c", subcore_axis_name="s")

    @functools.partial(
        pl.kernel,
        out_type=jax.ShapeDtypeStruct((B, D), jnp.float32),
        mesh=mesh,
        compiler_params=_SC_PARAMS,
        scratch_types=[
            pltpu.VMEM((NCHUNK, CHUNK), jnp.int32),
            [pltpu.VMEM((CHUNK, D), jnp.float32) for _ in range(NCHUNK)],
            pltpu.SemaphoreType.DMA,
        ],
    )
    def k(item_id_hbm, itab_hbm, iemb_hbm, ids_v, irows_v, sem):
        wid = lax.axis_index("s") * NC + lax.axis_index("c")
        base = wid * BPW
        for c in range(NCHUNK):
            pltpu.sync_copy(item_id_hbm.at[pl.ds(base + c * CHUNK, CHUNK)],
                            ids_v.at[c])
        descs = [pltpu.async_copy(itab_hbm.at[ids_v.at[c]], irows_v[c], sem)
                 for c in range(NCHUNK)]
        for c in range(NCHUNK):
            descs[c].wait()
            pltpu.sync_copy(irows_v[c],
                            iemb_hbm.at[pl.ds(base + c * CHUNK, CHUNK)])

    return k(item_id, item_table)


def _tc_mlp(i_emb, g_emb, cont8, wc8, bc, w1a, w1b, w1c, b1, w2, b2):
    BM = 1024
    grid = (B // BM,)

    def body(i_ref, g_ref, c_ref, wc_ref, bc_ref, w1a_ref, w1b_ref, w1c_ref,
             b1_ref, w2_ref, b2_ref, o_ref):
        ce = jnp.dot(c_ref[...], wc_ref[...], preferred_element_type=jnp.float32)
        ce = jnp.maximum(ce + bc_ref[...], 0.0)
        h = (jnp.dot(i_ref[...], w1a_ref[...], preferred_element_type=jnp.float32)
             + jnp.dot(g_ref[...], w1b_ref[...], preferred_element_type=jnp.float32)
             + jnp.dot(ce, w1c_ref[...], preferred_element_type=jnp.float32)
             + b1_ref[...])
        h = jnp.maximum(h, 0.0)
        out = jnp.dot(h, w2_ref[...], preferred_element_type=jnp.float32) + b2_ref[...]
        nrm = jnp.sqrt(jnp.sum(out * out, axis=1, keepdims=True))
        o_ref[...] = out / jnp.maximum(nrm, 1e-12)

    full = lambda shape: pl.BlockSpec(shape, lambda i: (0, 0))
    return pl.pallas_call(
        body,
        grid=grid,
        in_specs=[
            pl.BlockSpec((BM, D), lambda i: (i, 0)),
            pl.BlockSpec((BM, D), lambda i: (i, 0)),
            pl.BlockSpec((BM, 8), lambda i: (i, 0)),
            full((8, D)),
            full((1, D)),
            full((D, 128)),
            full((D, 128)),
            full((D, 128)),
            full((1, 128)),
            full((128, D)),
            full((1, D)),
        ],
        out_specs=pl.BlockSpec((BM, D), lambda i: (i, 0)),
        out_shape=jax.ShapeDtypeStruct((B, D), jnp.float32),
    )(i_emb, g_emb, cont8, wc8, bc, w1a, w1b, w1c, b1, w2, b2)


def kernel(item_id, tmdb_genres, release_year, avg_rating, revenue,
           item_table, genre_table, w_cont, b_cont, w1, b1, w2, b2):
    item_id = item_id.astype(jnp.int32)
    genres_flat = tmdb_genres.astype(jnp.int32).reshape(-1)

    g_emb = _sc_genre(genres_flat, genre_table)
    i_emb = _sc_item(item_id, item_table)

    cont = jnp.stack([release_year, avg_rating, revenue], axis=1)
    cont8 = jnp.pad(cont, ((0, 0), (0, 5)))
    wc8 = jnp.pad(w_cont, ((0, 5), (0, 0)))
    w1a, w1b, w1c = w1[:D], w1[D:2 * D], w1[2 * D:]

    return _tc_mlp(i_emb, g_emb, cont8, wc8, b_cont.reshape(1, D),
                   w1a, w1b, w1c, b1.reshape(1, 128), w2, b2.reshape(1, D))


# speedup vs baseline: 12.0331x; 1.5590x over previous
import functools

import jax
import jax.numpy as jnp
from jax import lax
from jax.experimental import pallas as pl
from jax.experimental.pallas import tpu as pltpu
from jax.experimental.pallas import tpu_sc as plsc

B = 16384
V = 1000001
G = 1001
D = 64
GL = 20

NC = 2
NS = 16
NW = NC * NS
BPW = B // NW
CHUNK = 128
NCHUNK = BPW // CHUNK
LANES = 16
NGRP = CHUNK // LANES

_SC_PARAMS = pltpu.CompilerParams(
    needs_layout_passes=False, use_tc_tiling_on_sc=False)


def _tree_sum(vals):
    vals = list(vals)
    while len(vals) > 1:
        nxt = [vals[i] + vals[i + 1] for i in range(0, len(vals) - 1, 2)]
        if len(vals) % 2:
            nxt.append(vals[-1])
        vals = nxt
    return vals[0]


def _sc_genre(genres_flat, genre_table):
    mesh = plsc.VectorSubcoreMesh(core_axis_name="c", subcore_axis_name="s")

    @functools.partial(
        pl.kernel,
        out_type=jax.ShapeDtypeStruct((B, D), jnp.float32),
        mesh=mesh,
        compiler_params=_SC_PARAMS,
        scratch_types=[
            pltpu.VMEM((G, D + 1), jnp.float32),
            pltpu.VMEM((CHUNK * GL,), jnp.int32),
            pltpu.VMEM((CHUNK, D + 1), jnp.float32),
        ],
    )
    def k(genres_hbm, gtab_hbm, gemb_hbm, gt_v, gids_v, g_v):
        wid = lax.axis_index("s") * NC + lax.axis_index("c")
        base = wid * BPW
        pltpu.sync_copy(gtab_hbm, gt_v.at[:, pl.ds(0, D)])
        iota = lax.iota(jnp.int32, LANES)

        for c in range(NCHUNK):
            cbase = base + c * CHUNK
            pltpu.sync_copy(genres_hbm.at[pl.ds(cbase * GL, CHUNK * GL)], gids_v)

            def grp_body(g, _):
                row16 = g * LANES + iota
                rowg = row16 * GL
                ids = [plsc.load_gather(gids_v, [rowg + l]) for l in range(GL)]
                one = jnp.full((LANES,), 1.0, jnp.float32)
                zero = jnp.full((LANES,), 0.0, jnp.float32)
                cnts = [jnp.where(ids[l] > 0, one, zero) for l in range(GL)]
                cnt = _tree_sum(cnts)
                recip = 1.0 / (cnt + 1e-8)

                @plsc.parallel_loop(0, D, step=1, unroll=4)
                def d_body(dd):
                    dvec = jnp.full((LANES,), dd, jnp.int32)
                    vals = [plsc.load_gather(gt_v, [ids[l], dvec])
                            for l in range(GL)]
                    acc = _tree_sum(vals)
                    plsc.store_scatter(g_v, [row16, dvec], acc * recip)

                return _

            lax.fori_loop(0, NGRP, grp_body, 0)
            pltpu.sync_copy(g_v.at[:, pl.ds(0, D)],
                            gemb_hbm.at[pl.ds(cbase, CHUNK)])

    return k(genres_flat, genre_table)


CB = 65536
NCB = 16
XP = CB * NCB
NFB = D // 8


def _tc_detile(itab_t):
    def body(in_ref, *out_refs):
        for r in range(8):
            out_refs[r][...] = in_ref[r, :]

    return pl.pallas_call(
        body,
        grid=(NFB, NCB),
        in_specs=[pl.BlockSpec((8, CB), lambda fb, c: (fb, c))],
        out_specs=tuple(
            pl.BlockSpec((CB,), lambda fb, c: (fb * NCB + c,)) for _ in range(8)),
        out_shape=tuple(
            jax.ShapeDtypeStruct((NFB * XP,), jnp.float32) for _ in range(8)),
    )(itab_t)


def _sc_item(item_id, tabs):
    mesh = plsc.VectorSubcoreMesh(core_axis_name="c", subcore_axis_name="s")
    FMS = CHUNK + 1

    @functools.partial(
        pl.kernel,
        out_type=jax.ShapeDtypeStruct((B * D,), jnp.float32),
        mesh=mesh,
        compiler_params=_SC_PARAMS,
        scratch_types=[
            pltpu.VMEM((NCHUNK, CHUNK), jnp.int32),
            pltpu.VMEM((D * CHUNK,), jnp.int32),
            pltpu.VMEM((D, FMS), jnp.float32),
            pltpu.VMEM((CHUNK * D,), jnp.float32),
            pltpu.SemaphoreType.DMA,
        ],
    )
    def k(item_id_hbm, t0, t1, t2, t3, t4, t5, t6, t7, iemb_hbm,
          ids_v, idx_v, fm_v, irows_v, sem):
        tab = [t0, t1, t2, t3, t4, t5, t6, t7]
        wid = lax.axis_index("s") * NC + lax.axis_index("c")
        base = wid * BPW
        for c in range(NCHUNK):
            pltpu.sync_copy(item_id_hbm.at[pl.ds(base + c * CHUNK, CHUNK)],
                            ids_v.at[c])
        iota = lax.iota(jnp.int32, LANES)

        for c in range(NCHUNK):
            cbase = base + c * CHUNK
            idg = [ids_v[c, pl.ds(g * LANES, LANES)] for g in range(NGRP)]

            @plsc.parallel_loop(0, D, step=1, unroll=2)
            def build(dd):
                off = jnp.full((LANES,), (dd // 8) * XP, jnp.int32)
                for g in range(NGRP):
                    idx_v[pl.ds(dd * CHUNK + g * LANES, LANES)] = idg[g] + off

            for r in range(8):
                def fire(fb, _):
                    dd = fb * 8 + r
                    pltpu.async_copy(
                        tab[r].at[idx_v.at[pl.ds(dd * CHUNK, CHUNK)]],
                        fm_v.at[dd, pl.ds(0, CHUNK)], sem)
                    return _

                lax.fori_loop(0, NFB, fire, 0)
            pltpu.make_async_copy(
                t0.at[pl.ds(0, CHUNK * D)], irows_v, sem).wait()

            @plsc.parallel_loop(0, CHUNK, step=1, unroll=2)
            def trans(j):
                jvec = jnp.full((LANES,), j, jnp.int32)
                for g in range(D // LANES):
                    v = plsc.load_gather(fm_v, [g * LANES + iota, jvec])
                    irows_v[pl.ds(j * D + g * LANES, LANES)] = v

            pltpu.sync_copy(irows_v, iemb_hbm.at[pl.ds(cbase * D, CHUNK * D)])

    return k(item_id, *tabs)


def _tc_mlp(i_emb, g_emb, cont8, wc8, bc, w1a, w1b, w1c, b1, w2, b2):
    BM = 1024
    grid = (B // BM,)

    def body(i_ref, g_ref, c_ref, wc_ref, bc_ref, w1a_ref, w1b_ref, w1c_ref,
             b1_ref, w2_ref, b2_ref, o_ref):
        ce = jnp.dot(c_ref[...], wc_ref[...], preferred_element_type=jnp.float32)
        ce = jnp.maximum(ce + bc_ref[...], 0.0)
        h = (jnp.dot(i_ref[...], w1a_ref[...], preferred_element_type=jnp.float32)
             + jnp.dot(g_ref[...], w1b_ref[...], preferred_element_type=jnp.float32)
             + jnp.dot(ce, w1c_ref[...], preferred_element_type=jnp.float32)
             + b1_ref[...])
        h = jnp.maximum(h, 0.0)
        out = jnp.dot(h, w2_ref[...], preferred_element_type=jnp.float32) + b2_ref[...]
        nrm = jnp.sqrt(jnp.sum(out * out, axis=1, keepdims=True))
        o_ref[...] = out / jnp.maximum(nrm, 1e-12)

    full = lambda shape: pl.BlockSpec(shape, lambda i: (0, 0))
    return pl.pallas_call(
        body,
        grid=grid,
        in_specs=[
            pl.BlockSpec((BM, D), lambda i: (i, 0)),
            pl.BlockSpec((BM, D), lambda i: (i, 0)),
            pl.BlockSpec((BM, 8), lambda i: (i, 0)),
            full((8, D)),
            full((1, D)),
            full((D, 128)),
            full((D, 128)),
            full((D, 128)),
            full((1, 128)),
            full((128, D)),
            full((1, D)),
        ],
        out_specs=pl.BlockSpec((BM, D), lambda i: (i, 0)),
        out_shape=jax.ShapeDtypeStruct((B, D), jnp.float32),
    )(i_emb, g_emb, cont8, wc8, bc, w1a, w1b, w1c, b1, w2, b2)


def kernel(item_id, tmdb_genres, release_year, avg_rating, revenue,
           item_table, genre_table, w_cont, b_cont, w1, b1, w2, b2):
    item_id = item_id.astype(jnp.int32)
    genres_flat = tmdb_genres.astype(jnp.int32).reshape(-1)

    g_emb = _sc_genre(genres_flat, genre_table)
    tabs = _tc_detile(item_table.T)
    i_emb = _sc_item(item_id, tabs).reshape(B, D)

    cont = jnp.stack([release_year, avg_rating, revenue], axis=1)
    cont8 = jnp.pad(cont, ((0, 0), (0, 5)))
    wc8 = jnp.pad(w_cont, ((0, 5), (0, 0)))
    w1a, w1b, w1c = w1[:D], w1[D:2 * D], w1[2 * D:]

    return _tc_mlp(i_emb, g_emb, cont8, wc8, b_cont.reshape(1, D),
                   w1a, w1b, w1c, b1.reshape(1, 128), w2, b2.reshape(1, D))


# speedup vs baseline: 12.0605x; 1.0023x over previous
import functools

import jax
import jax.numpy as jnp
from jax import lax
from jax.experimental import pallas as pl
from jax.experimental.pallas import tpu as pltpu
from jax.experimental.pallas import tpu_sc as plsc

B = 16384
V = 1000001
G = 1001
D = 64
GL = 20

NC = 2
NS = 16
NW = NC * NS
BPW = B // NW
CHUNK = 128
NCHUNK = BPW // CHUNK
LANES = 16
NGRP = CHUNK // LANES

_SC_PARAMS = pltpu.CompilerParams(
    needs_layout_passes=False, use_tc_tiling_on_sc=False)


def _tree_sum(vals):
    vals = list(vals)
    while len(vals) > 1:
        nxt = [vals[i] + vals[i + 1] for i in range(0, len(vals) - 1, 2)]
        if len(vals) % 2:
            nxt.append(vals[-1])
        vals = nxt
    return vals[0]


CB = 65536
NCB = 16
XP = CB * NCB
NFB = D // 8


def _tc_detile(itab_t):
    def body(in_ref, *out_refs):
        for r in range(8):
            out_refs[r][...] = in_ref[r, :]

    return pl.pallas_call(
        body,
        grid=(NFB, NCB),
        in_specs=[pl.BlockSpec((8, CB), lambda fb, c: (fb, c))],
        out_specs=tuple(
            pl.BlockSpec((CB,), lambda fb, c: (fb * NCB + c,)) for _ in range(8)),
        out_shape=tuple(
            jax.ShapeDtypeStruct((NFB * XP,), jnp.float32) for _ in range(8)),
    )(itab_t)


def _sc_item_genre(item_id, tabs, genres_flat, genre_table):
    mesh = plsc.VectorSubcoreMesh(core_axis_name="c", subcore_axis_name="s")
    FMS = CHUNK + 1

    @functools.partial(
        pl.kernel,
        out_type=(jax.ShapeDtypeStruct((B * D,), jnp.float32),
                  jax.ShapeDtypeStruct((B, D), jnp.float32)),
        mesh=mesh,
        compiler_params=_SC_PARAMS,
        scratch_types=[
            pltpu.VMEM((NCHUNK, CHUNK), jnp.int32),
            pltpu.VMEM((D * CHUNK,), jnp.int32),
            pltpu.VMEM((D, FMS), jnp.float32),
            pltpu.VMEM((CHUNK * D,), jnp.float32),
            pltpu.VMEM((G, D + 1), jnp.float32),
            pltpu.VMEM((CHUNK * GL,), jnp.int32),
            pltpu.VMEM((CHUNK, D + 1), jnp.float32),
            pltpu.SemaphoreType.DMA,
        ],
    )
    def k(item_id_hbm, t0, t1, t2, t3, t4, t5, t6, t7, genres_hbm, gtab_hbm,
          iemb_hbm, gemb_hbm, ids_v, idx_v, fm_v, irows_v, gt_v, gids_v, g_v,
          sem):
        tab = [t0, t1, t2, t3, t4, t5, t6, t7]
        wid = lax.axis_index("s") * NC + lax.axis_index("c")
        base = wid * BPW
        pltpu.sync_copy(gtab_hbm, gt_v.at[:, pl.ds(0, D)])
        for c in range(NCHUNK):
            pltpu.sync_copy(item_id_hbm.at[pl.ds(base + c * CHUNK, CHUNK)],
                            ids_v.at[c])
        iota = lax.iota(jnp.int32, LANES)

        for c in range(NCHUNK):
            cbase = base + c * CHUNK
            idg = [ids_v[c, pl.ds(g * LANES, LANES)] for g in range(NGRP)]

            @plsc.parallel_loop(0, D, step=1, unroll=2)
            def build(dd):
                off = jnp.full((LANES,), (dd // 8) * XP, jnp.int32)
                for g in range(NGRP):
                    idx_v[pl.ds(dd * CHUNK + g * LANES, LANES)] = idg[g] + off

            for r in range(8):
                def fire(fb, _):
                    dd = fb * 8 + r
                    pltpu.async_copy(
                        tab[r].at[idx_v.at[pl.ds(dd * CHUNK, CHUNK)]],
                        fm_v.at[dd, pl.ds(0, CHUNK)], sem)
                    return _

                lax.fori_loop(0, NFB, fire, 0)

            pltpu.sync_copy(genres_hbm.at[pl.ds(cbase * GL, CHUNK * GL)], gids_v)

            def grp_body(g, _):
                row16 = g * LANES + iota
                rowg = row16 * GL
                ids = [plsc.load_gather(gids_v, [rowg + l]) for l in range(GL)]
                one = jnp.full((LANES,), 1.0, jnp.float32)
                zero = jnp.full((LANES,), 0.0, jnp.float32)
                cnt = _tree_sum([jnp.where(ids[l] > 0, one, zero)
                                 for l in range(GL)])
                recip = 1.0 / (cnt + 1e-8)

                @plsc.parallel_loop(0, D, step=1, unroll=4)
                def d_body(dd):
                    dvec = jnp.full((LANES,), dd, jnp.int32)
                    vals = [plsc.load_gather(gt_v, [ids[l], dvec])
                            for l in range(GL)]
                    plsc.store_scatter(g_v, [row16, dvec],
                                       _tree_sum(vals) * recip)

                return _

            lax.fori_loop(0, NGRP, grp_body, 0)
            pltpu.sync_copy(g_v.at[:, pl.ds(0, D)],
                            gemb_hbm.at[pl.ds(cbase, CHUNK)])

            pltpu.make_async_copy(
                t0.at[pl.ds(0, CHUNK * D)], irows_v, sem).wait()

            @plsc.parallel_loop(0, CHUNK, step=1, unroll=2)
            def trans(j):
                jvec = jnp.full((LANES,), j, jnp.int32)
                for g in range(D // LANES):
                    v = plsc.load_gather(fm_v, [g * LANES + iota, jvec])
                    irows_v[pl.ds(j * D + g * LANES, LANES)] = v

            pltpu.sync_copy(irows_v, iemb_hbm.at[pl.ds(cbase * D, CHUNK * D)])

    return k(item_id, *tabs, genres_flat, genre_table)


def _tc_mlp(i_emb, g_emb, cont8, wc8, bc, w1a, w1b, w1c, b1, w2, b2):
    BM = 1024
    grid = (B // BM,)

    def body(i_ref, g_ref, c_ref, wc_ref, bc_ref, w1a_ref, w1b_ref, w1c_ref,
             b1_ref, w2_ref, b2_ref, o_ref):
        ce = jnp.dot(c_ref[...], wc_ref[...], preferred_element_type=jnp.float32)
        ce = jnp.maximum(ce + bc_ref[...], 0.0)
        h = (jnp.dot(i_ref[...], w1a_ref[...], preferred_element_type=jnp.float32)
             + jnp.dot(g_ref[...], w1b_ref[...], preferred_element_type=jnp.float32)
             + jnp.dot(ce, w1c_ref[...], preferred_element_type=jnp.float32)
             + b1_ref[...])
        h = jnp.maximum(h, 0.0)
        out = jnp.dot(h, w2_ref[...], preferred_element_type=jnp.float32) + b2_ref[...]
        nrm = jnp.sqrt(jnp.sum(out * out, axis=1, keepdims=True))
        o_ref[...] = out / jnp.maximum(nrm, 1e-12)

    full = lambda shape: pl.BlockSpec(shape, lambda i: (0, 0))
    return pl.pallas_call(
        body,
        grid=grid,
        in_specs=[
            pl.BlockSpec((BM, D), lambda i: (i, 0)),
            pl.BlockSpec((BM, D), lambda i: (i, 0)),
            pl.BlockSpec((BM, 8), lambda i: (i, 0)),
            full((8, D)),
            full((1, D)),
            full((D, 128)),
            full((D, 128)),
            full((D, 128)),
            full((1, 128)),
            full((128, D)),
            full((1, D)),
        ],
        out_specs=pl.BlockSpec((BM, D), lambda i: (i, 0)),
        out_shape=jax.ShapeDtypeStruct((B, D), jnp.float32),
    )(i_emb, g_emb, cont8, wc8, bc, w1a, w1b, w1c, b1, w2, b2)


def kernel(item_id, tmdb_genres, release_year, avg_rating, revenue,
           item_table, genre_table, w_cont, b_cont, w1, b1, w2, b2):
    item_id = item_id.astype(jnp.int32)
    genres_flat = tmdb_genres.astype(jnp.int32).reshape(-1)

    tabs = _tc_detile(item_table.T)
    i_emb_flat, g_emb = _sc_item_genre(item_id, tabs, genres_flat, genre_table)
    i_emb = i_emb_flat.reshape(B, D)

    cont = jnp.stack([release_year, avg_rating, revenue], axis=1)
    cont8 = jnp.pad(cont, ((0, 0), (0, 5)))
    wc8 = jnp.pad(w_cont, ((0, 5), (0, 0)))
    w1a, w1b, w1c = w1[:D], w1[D:2 * D], w1[2 * D:]

    return _tc_mlp(i_emb, g_emb, cont8, wc8, b_cont.reshape(1, D),
                   w1a, w1b, w1c, b1.reshape(1, 128), w2, b2.reshape(1, D))
